# baseline (speedup 1.0000x reference)
import functools

import jax
import jax.numpy as jnp
from jax import lax
from jax.experimental import pallas as pl
from jax.experimental.pallas import tpu as pltpu
from jax.experimental.pallas import tpu_sc as plsc

_NUM_CORES = 2
_NUM_SUBCORES = 16
_NUM_WORKERS = _NUM_CORES * _NUM_SUBCORES
_LANES = 16

_OUT_CHUNK = 2048


def _sc_gather(x_flat, w):
    d_model, vocab = w.shape
    n = x_flat.shape[0]
    rows_per_worker = d_model // _NUM_WORKERS

    mesh = plsc.VectorSubcoreMesh(core_axis_name="c", subcore_axis_name="s")

    @functools.partial(
        pl.kernel,
        out_type=jax.ShapeDtypeStruct((d_model, n), jnp.float32),
        mesh=mesh,
        scratch_types=[
            pltpu.VMEM((vocab,), jnp.float32),
            pltpu.VMEM((n,), jnp.int32),
            pltpu.VMEM((_OUT_CHUNK,), jnp.float32),
        ],
        compiler_params=pltpu.CompilerParams(needs_layout_passes=False),
    )
    def sc_kernel(x_hbm, w_hbm, out_hbm, row_v, idx_v, out_v):
        wid = lax.axis_index("s") * _NUM_CORES + lax.axis_index("c")
        pltpu.sync_copy(x_hbm, idx_v)

        def row_body(r, carry):
            d = wid * rows_per_worker + r
            pltpu.sync_copy(w_hbm.at[d], row_v)

            def chunk_body(cb, carry2):
                base = cb * _OUT_CHUNK
                for i in range(_OUT_CHUNK // _LANES):
                    idx16 = idx_v[pl.ds(base + i * _LANES, _LANES)]
                    out_v[pl.ds(i * _LANES, _LANES)] = plsc.load_gather(
                        row_v, [idx16])
                pltpu.sync_copy(out_v, out_hbm.at[d, pl.ds(base, _OUT_CHUNK)])
                return carry2

            return lax.fori_loop(0, n // _OUT_CHUNK, chunk_body, carry)

        lax.fori_loop(0, rows_per_worker, row_body, 0)

    return sc_kernel(x_flat, w)


def _tc_transpose(gathered):
    d_model, n = gathered.shape
    bd, bn = 512, 512

    def tp_body(in_ref, out_ref):
        out_ref[...] = in_ref[...].T

    return pl.pallas_call(
        tp_body,
        grid=(d_model // bd, n // bn),
        in_specs=[pl.BlockSpec((bd, bn), lambda i, j: (i, j))],
        out_specs=pl.BlockSpec((bn, bd), lambda i, j: (j, i)),
        out_shape=jax.ShapeDtypeStruct((n, d_model), jnp.float32),
    )(gathered)


def kernel(x, W_embed):
    b, p = x.shape
    d_model = W_embed.shape[0]
    x_flat = x.reshape(b * p).astype(jnp.int32)
    gathered = _sc_gather(x_flat, W_embed)
    out = _tc_transpose(gathered)
    return out.reshape(b, p, d_model)
